# baseline (speedup 1.0000x reference)
import jax
import jax.numpy as jnp
from jax import lax
from jax.experimental import pallas as pl
from jax.experimental.pallas import tpu as pltpu
from jax.experimental.pallas import tpu_sc as plsc

NUM_EMBEDDINGS = 1000000
EMBEDDING_DIM = 32
BATCH = 4096
SEQ_LEN = 200

_B = BATCH * SEQ_LEN
_NC = 2
_NS = 16
_NW = _NC * _NS
_PER_W = _B // _NW
_BPC = 2
_CHUNK = _BPC * SEQ_LEN
_NCHUNK = _PER_W // _CHUNK
_TSIZE = _CHUNK * EMBEDDING_DIM


def _body(x_hbm, w_hbm, out_hbm, idx_v, rows_v, trans_v, gsems, wsems):
    wid = lax.axis_index("s") * _NC + lax.axis_index("c")
    base = wid * _PER_W
    obase = wid * (_PER_W * EMBEDDING_DIM)
    iot = lax.iota(jnp.int32, 16)

    def start_gather(k, pb):
        pltpu.make_async_copy(
            w_hbm.at[idx_v.at[pl.ds(k * _CHUNK, _CHUNK)]],
            rows_v.at[pb],
            gsems.at[pb],
        ).start()

    def wait_gather(pb):
        pltpu.make_async_copy(
            w_hbm.at[idx_v.at[pl.ds(0, _CHUNK)]], rows_v.at[pb], gsems.at[pb]
        ).wait()

    def start_write(k, pb):
        pltpu.make_async_copy(
            trans_v.at[pb],
            out_hbm.at[pl.ds(obase + k * _TSIZE, _TSIZE)],
            wsems.at[pb],
        ).start()

    def wait_write(pb):
        pltpu.make_async_copy(
            trans_v.at[pb], out_hbm.at[pl.ds(obase, _TSIZE)], wsems.at[pb]
        ).wait()

    def transpose(pb):
        def blk(b, carry):
            j0 = jnp.minimum(b * 16, SEQ_LEN - 16)
            rbase = j0 + iot
            for bi in range(_BPC):
                ridx = rbase + bi * SEQ_LEN
                for c in range(EMBEDDING_DIM):
                    col = jnp.full((16,), c, jnp.int32)
                    vals = plsc.load_gather(rows_v.at[pb], [ridx, col])
                    off = bi * (SEQ_LEN * EMBEDDING_DIM) + c * SEQ_LEN
                    trans_v[pb, pl.ds(off + j0, 16)] = vals
            return carry

        lax.fori_loop(0, 13, blk, 0)

    pltpu.sync_copy(x_hbm.at[pl.ds(base, _PER_W)], idx_v)

    start_gather(0, 0)
    start_gather(1, 1)
    for k in range(2):
        wait_gather(k)
        transpose(k)
        start_gather(k + 2, k)
        start_write(k, k)

    def steady(p, carry):
        for b in range(2):
            k = 2 * p + b
            wait_gather(b)
            wait_write(b)
            transpose(b)
            start_gather(k + 2, b)
            start_write(k, b)
        return carry

    lax.fori_loop(1, _NCHUNK // 2 - 1, steady, 0)

    for b in range(2):
        k = _NCHUNK - 2 + b
        wait_gather(b)
        wait_write(b)
        transpose(b)
        start_write(k, b)
    wait_write(0)
    wait_write(1)


@jax.jit
def _run(x_flat, weight):
    mesh = plsc.VectorSubcoreMesh(core_axis_name="c", subcore_axis_name="s")
    return pl.kernel(
        _body,
        out_type=jax.ShapeDtypeStruct((_B * EMBEDDING_DIM,), jnp.float32),
        mesh=mesh,
        scratch_types=[
            pltpu.VMEM((_PER_W,), jnp.int32),
            pltpu.VMEM((2, _CHUNK, EMBEDDING_DIM), jnp.float32),
            pltpu.VMEM((2, _TSIZE), jnp.float32),
            pltpu.SemaphoreType.DMA((2,)),
            pltpu.SemaphoreType.DMA((2,)),
        ],
        compiler_params=pltpu.CompilerParams(
            use_tc_tiling_on_sc=False, needs_layout_passes=False
        ),
    )(x_flat, weight)


def kernel(x, weight):
    flat = _run(x.reshape(-1), weight)
    out3 = flat.reshape(BATCH, EMBEDDING_DIM, SEQ_LEN)
    return out3.transpose(0, 2, 1)


# speedup vs baseline: 1.2117x; 1.2117x over previous
import jax
import jax.numpy as jnp
from jax import lax
from jax.experimental import pallas as pl
from jax.experimental.pallas import tpu as pltpu
from jax.experimental.pallas import tpu_sc as plsc

NUM_EMBEDDINGS = 1000000
EMBEDDING_DIM = 32
BATCH = 4096
SEQ_LEN = 200

_B = BATCH * SEQ_LEN
_NC = 2
_NS = 16
_NW = _NC * _NS
_PER_W = _B // _NW
_BPC = 2
_CHUNK = _BPC * SEQ_LEN
_NCHUNK = _PER_W // _CHUNK
_TSIZE = _CHUNK * EMBEDDING_DIM


def _body(x_hbm, w_hbm, out_hbm, idx_v, rows_v, trans_v, gsems, wsems):
    wid = lax.axis_index("s") * _NC + lax.axis_index("c")
    base = wid * _PER_W
    obase = wid * (_PER_W * EMBEDDING_DIM)
    iot = lax.iota(jnp.int32, 16)

    def start_gather(k, pb):
        pltpu.make_async_copy(
            w_hbm.at[idx_v.at[pl.ds(k * _CHUNK, _CHUNK)]],
            rows_v.at[pb],
            gsems.at[pb],
        ).start()

    def wait_gather(pb):
        pltpu.make_async_copy(
            w_hbm.at[idx_v.at[pl.ds(0, _CHUNK)]], rows_v.at[pb], gsems.at[pb]
        ).wait()

    def start_write(k, pb):
        pltpu.make_async_copy(
            trans_v.at[pb],
            out_hbm.at[pl.ds(obase + k * _TSIZE, _TSIZE)],
            wsems.at[pb],
        ).start()

    def wait_write(pb):
        pltpu.make_async_copy(
            trans_v.at[pb], out_hbm.at[pl.ds(obase, _TSIZE)], wsems.at[pb]
        ).wait()

    iot32 = iot * EMBEDDING_DIM

    def transpose(pb):
        @plsc.parallel_loop(0, 13 * 2 * EMBEDDING_DIM, unroll=8)
        def _(q):
            blk = q >> 6
            g = q & 63
            bi = g >> 5
            c = g & 31
            j0 = jnp.minimum(blk * 16, SEQ_LEN - 16)
            ridx = (j0 + bi * SEQ_LEN) + iot
            col = jnp.full((16,), 0, jnp.int32) + c
            vals = plsc.load_gather(rows_v.at[pb], [ridx, col])
            off = bi * (SEQ_LEN * EMBEDDING_DIM) + c * SEQ_LEN + j0
            trans_v[pb, pl.ds(off, 16)] = vals

    pltpu.sync_copy(x_hbm.at[pl.ds(base, _PER_W)], idx_v)

    start_gather(0, 0)
    start_gather(1, 1)
    for k in range(2):
        wait_gather(k)
        transpose(k)
        start_gather(k + 2, k)
        start_write(k, k)

    def steady(p, carry):
        for b in range(2):
            k = 2 * p + b
            wait_gather(b)
            wait_write(b)
            transpose(b)
            start_gather(k + 2, b)
            start_write(k, b)
        return carry

    lax.fori_loop(1, _NCHUNK // 2 - 1, steady, 0)

    for b in range(2):
        k = _NCHUNK - 2 + b
        wait_gather(b)
        wait_write(b)
        transpose(b)
        start_write(k, b)
    wait_write(0)
    wait_write(1)


@jax.jit
def _run(x_flat, weight):
    mesh = plsc.VectorSubcoreMesh(core_axis_name="c", subcore_axis_name="s")
    return pl.kernel(
        _body,
        out_type=jax.ShapeDtypeStruct((_B * EMBEDDING_DIM,), jnp.float32),
        mesh=mesh,
        scratch_types=[
            pltpu.VMEM((_PER_W,), jnp.int32),
            pltpu.VMEM((2, _CHUNK, EMBEDDING_DIM), jnp.float32),
            pltpu.VMEM((2, _TSIZE), jnp.float32),
            pltpu.SemaphoreType.DMA((2,)),
            pltpu.SemaphoreType.DMA((2,)),
        ],
        compiler_params=pltpu.CompilerParams(
            use_tc_tiling_on_sc=False, needs_layout_passes=False
        ),
    )(x_flat, weight)


def kernel(x, weight):
    flat = _run(x.reshape(-1), weight)
    out3 = flat.reshape(BATCH, EMBEDDING_DIM, SEQ_LEN)
    return out3.transpose(0, 2, 1)


# speedup vs baseline: 1.6534x; 1.3645x over previous
import jax
import jax.numpy as jnp
from jax import lax
from jax.experimental import pallas as pl
from jax.experimental.pallas import tpu as pltpu
from jax.experimental.pallas import tpu_sc as plsc

NUM_EMBEDDINGS = 1000000
EMBEDDING_DIM = 32
BATCH = 4096
SEQ_LEN = 200

_B = BATCH * SEQ_LEN
_NC = 2
_NS = 16
_NW = _NC * _NS
_PER_W = _B // _NW
_BPC = 2
_CHUNK = _BPC * SEQ_LEN
_NCHUNK = _PER_W // _CHUNK
_TSIZE = _CHUNK * EMBEDDING_DIM


def _body(x_hbm, w_hbm, out_hbm, idx_v, rows_v, trans_v, gsems, wsems):
    wid = lax.axis_index("s") * _NC + lax.axis_index("c")
    base = wid * _PER_W
    obase = wid * (_PER_W * EMBEDDING_DIM)
    iot = lax.iota(jnp.int32, 16)

    def start_gather(k, pb):
        pltpu.make_async_copy(
            w_hbm.at[idx_v.at[pl.ds(k * _CHUNK, _CHUNK)]],
            rows_v.at[pb],
            gsems.at[pb],
        ).start()

    def wait_gather(pb):
        pltpu.make_async_copy(
            w_hbm.at[idx_v.at[pl.ds(0, _CHUNK)]], rows_v.at[pb], gsems.at[pb]
        ).wait()

    def start_write(k, pb):
        pltpu.make_async_copy(
            trans_v.at[pb],
            out_hbm.at[pl.ds(obase + k * _TSIZE, _TSIZE)],
            wsems.at[pb],
        ).start()

    def wait_write(pb):
        pltpu.make_async_copy(
            trans_v.at[pb], out_hbm.at[pl.ds(obase, _TSIZE)], wsems.at[pb]
        ).wait()

    cidx0 = iot * SEQ_LEN
    cidx1 = cidx0 + 16 * SEQ_LEN

    def transpose(pb):
        @plsc.parallel_loop(0, _CHUNK, unroll=8)
        def _(r):
            dbase = r + jnp.where(r >= SEQ_LEN,
                                  (SEQ_LEN * EMBEDDING_DIM) - SEQ_LEN, 0)
            v0 = rows_v[pb, r, pl.ds(0, 16)]
            v1 = rows_v[pb, r, pl.ds(16, 16)]
            plsc.store_scatter(trans_v.at[pb], [cidx0 + dbase], v0)
            plsc.store_scatter(trans_v.at[pb], [cidx1 + dbase], v1)

    pltpu.sync_copy(x_hbm.at[pl.ds(base, _PER_W)], idx_v)

    start_gather(0, 0)
    start_gather(1, 1)
    for k in range(2):
        wait_gather(k)
        transpose(k)
        start_gather(k + 2, k)
        start_write(k, k)

    def steady(p, carry):
        for b in range(2):
            k = 2 * p + b
            wait_gather(b)
            wait_write(b)
            transpose(b)
            start_gather(k + 2, b)
            start_write(k, b)
        return carry

    lax.fori_loop(1, _NCHUNK // 2 - 1, steady, 0)

    for b in range(2):
        k = _NCHUNK - 2 + b
        wait_gather(b)
        wait_write(b)
        transpose(b)
        start_write(k, b)
    wait_write(0)
    wait_write(1)


@jax.jit
def _run(x_flat, weight):
    mesh = plsc.VectorSubcoreMesh(core_axis_name="c", subcore_axis_name="s")
    return pl.kernel(
        _body,
        out_type=jax.ShapeDtypeStruct((_B * EMBEDDING_DIM,), jnp.float32),
        mesh=mesh,
        scratch_types=[
            pltpu.VMEM((_PER_W,), jnp.int32),
            pltpu.VMEM((2, _CHUNK, EMBEDDING_DIM), jnp.float32),
            pltpu.VMEM((2, _TSIZE), jnp.float32),
            pltpu.SemaphoreType.DMA((2,)),
            pltpu.SemaphoreType.DMA((2,)),
        ],
        compiler_params=pltpu.CompilerParams(
            use_tc_tiling_on_sc=False, needs_layout_passes=False
        ),
    )(x_flat, weight)


def kernel(x, weight):
    flat = _run(x.reshape(-1), weight)
    out3 = flat.reshape(BATCH, EMBEDDING_DIM, SEQ_LEN)
    return out3.transpose(0, 2, 1)
